# Initial kernel scaffold; baseline (speedup 1.0000x reference)
import jax
import jax.numpy as jnp
from jax.experimental import pallas as pl


def kernel(features1, features2, x1, x2, nuv1, nuv2, topk, q1_w, q1_b, q2_w, q2_b, g1_w, g1_b, g2_w, g2_b, k1_w, k2_w, v1_w, v2_w, d1_w, d2_w):
    raise NotImplementedError("write your pallas kernel here")


# speedup vs baseline: 1.5418x; 1.5418x over previous
import functools

import jax
import jax.numpy as jnp
from jax import lax
from jax.experimental import pallas as pl
from jax.experimental.pallas import tpu as pltpu
from jax.experimental.pallas import tpu_sc as plsc

N1 = 50000
N2 = 50000
NN = 16
NI = 128
ND = 32
NH = 4

DT = 48
NW = 32
CH = 128
NCH = 196
EPW = NCH * CH
E = N1 * NN
E_PAD = NW * EPW
B = 400


def _sc_gather(table, idx3):
    mesh = plsc.VectorSubcoreMesh(core_axis_name="c", subcore_axis_name="s")

    @functools.partial(
        pl.kernel,
        mesh=mesh,
        compiler_params=pltpu.CompilerParams(use_tc_tiling_on_sc=False),
        out_type=jax.ShapeDtypeStruct((E_PAD, DT), jnp.float32),
        scratch_types=[
            pltpu.VMEM((NCH, CH), jnp.int32),
            pltpu.VMEM((2, CH, DT), jnp.float32),
            pltpu.SemaphoreType.DMA,
            pltpu.SemaphoreType.DMA,
        ],
    )
    def k(table_hbm, idx_hbm, out_hbm, idx_v, rows_v, sem_a, sem_b):
        wid = lax.axis_index("s") * 2 + lax.axis_index("c")
        base = wid * EPW
        pltpu.sync_copy(idx_hbm.at[wid], idx_v)
        pltpu.async_copy(table_hbm.at[idx_v.at[0]], rows_v.at[0], sem_a)

        def body(jj, carry):
            j0 = 2 * jj
            pltpu.async_copy(table_hbm.at[idx_v.at[j0 + 1]], rows_v.at[1], sem_b)
            pltpu.make_async_copy(
                table_hbm.at[idx_v.at[j0]], rows_v.at[0], sem_a).wait()
            pltpu.sync_copy(rows_v.at[0], out_hbm.at[pl.ds(base + j0 * CH, CH)])

            @pl.when(j0 + 2 < NCH)
            def _():
                pltpu.async_copy(
                    table_hbm.at[idx_v.at[j0 + 2]], rows_v.at[0], sem_a)

            pltpu.make_async_copy(
                table_hbm.at[idx_v.at[j0 + 1]], rows_v.at[1], sem_b).wait()
            pltpu.sync_copy(
                rows_v.at[1], out_hbm.at[pl.ds(base + (j0 + 1) * CH, CH)])
            return carry

        lax.fori_loop(0, NCH // 2, body, 0)

    return k(table, idx3)


def _elu(x):
    xn = jnp.minimum(x, 0.0)
    e = jnp.exp(xn)
    em1 = jnp.where(e == 1.0, xn, (e - 1.0) * xn / jnp.log(e))
    return jnp.where(x > 0, x, em1)


def _dot(a, b):
    return lax.dot_general(a, b, (((1,), (0,)), ((), ())),
                           preferred_element_type=jnp.float32)


def _dense_body(f1_ref, x1_ref, nuv_ref, tk_ref, g_ref,
                q1w_ref, q1b_ref, q2w_ref, q2b_ref, g1w_ref, g1b_ref,
                g2w_ref, g2b_ref, k1_ref, k2_ref, v1_ref, v2_ref,
                d1_ref, d2_ref, out_ref):
    h = _elu(_dot(f1_ref[...], q1w_ref[...]) + q1b_ref[...])
    Q = _dot(h, q2w_ref[...]) + q2b_ref[...]

    g = g_ref[...]
    nuv = nuv_ref[...]
    x1 = x1_ref[...]

    dx = [g[:, :, 32 + m] - x1[:, m][:, None] for m in range(3)]
    nv = [g[:, :, 35 + p] for p in range(9)]

    cols = []
    for kk in range(3):
        acc = nuv[:, 3 * kk][:, None] * dx[0]
        acc += nuv[:, 3 * kk + 1][:, None] * dx[1]
        acc += nuv[:, 3 * kk + 2][:, None] * dx[2]
        cols.append(acc)
    for r in range(3):
        for kk in range(3):
            acc = nuv[:, 3 * kk][:, None] * nv[3 * r]
            acc += nuv[:, 3 * kk + 1][:, None] * nv[3 * r + 1]
            acc += nuv[:, 3 * kk + 2][:, None] * nv[3 * r + 2]
            cols.append(acc)
    RL = jnp.stack(cols, axis=-1)

    d2 = dx[0] * dx[0] + dx[1] * dx[1] + dx[2] * dx[2]
    wgt = jnp.exp(-0.5 * d2)

    geo1 = _elu(_dot(RL.reshape(B * NN, 12), g1w_ref[...]) + g1b_ref[...])
    geo = _dot(geo1, g2w_ref[...]) + g2b_ref[...]
    geo3 = geo.reshape(B, NN, ND) * g[:, :, 0:32] * wgt[:, :, None]
    geo = geo3.reshape(B * NN, ND)

    K2 = _dot(_elu(_dot(geo, k1_ref[...])), k2_ref[...])
    V2 = _dot(_elu(_dot(geo, v1_ref[...])), v2_ref[...])
    K3 = K2.reshape(B, NN, ND)
    V3 = V2.reshape(B, NN, ND)

    msk = tk_ref[...] == 0
    outs = []
    for hh in range(NH):
        Qh = Q[:, ND * hh:ND * (hh + 1)]
        Mqh = jnp.sum(K3 * Qh[:, None, :], axis=2)
        Mqh = jnp.where(msk, 0.0, Mqh)
        outs.append(jnp.sum(Mqh[:, :, None] * V3, axis=1))
    o = jnp.concatenate(outs, axis=1)
    out_ref[...] = _elu(_dot(_elu(_dot(o, d1_ref[...])), d2_ref[...]))


def _row_spec(*shape):
    return pl.BlockSpec(shape, lambda b: (b,) + (0,) * (len(shape) - 1))


def _w_spec(*shape):
    return pl.BlockSpec(shape, lambda b: (0,) * len(shape))


def _tc_dense(f1, x1, nuv9, topk, g3, q1w, q1b, q2w, q2b, g1w, g1b,
              g2w, g2b, k1w, k2w, v1w, v2w, d1w, d2w):
    grid = (N1 // B,)
    in_specs = [
        _row_spec(B, NI),
        _row_spec(B, 3),
        _row_spec(B, 9),
        _row_spec(B, NN),
        _row_spec(B, NN, DT),
        _w_spec(NI, ND), _w_spec(1, ND), _w_spec(ND, NH * ND), _w_spec(1, NH * ND),
        _w_spec(12, ND), _w_spec(1, ND), _w_spec(ND, ND), _w_spec(1, ND),
        _w_spec(ND, ND), _w_spec(ND, ND), _w_spec(ND, ND), _w_spec(ND, ND),
        _w_spec(NH * ND, ND), _w_spec(ND, ND),
    ]
    return pl.pallas_call(
        _dense_body,
        grid=grid,
        in_specs=in_specs,
        out_specs=_row_spec(B, ND),
        out_shape=jax.ShapeDtypeStruct((N1, ND), jnp.float32),
    )(f1, x1, nuv9, topk, g3, q1w, q1b, q2w, q2b, g1w, g1b,
      g2w, g2b, k1w, k2w, v1w, v2w, d1w, d2w)


def kernel(features1, features2, x1, x2, nuv1, nuv2, topk,
           q1_w, q1_b, q2_w, q2_b, g1_w, g1_b, g2_w, g2_b,
           k1_w, k2_w, v1_w, v2_w, d1_w, d2_w):
    table = jnp.concatenate(
        [features2, x2, nuv2.reshape(N2, 9),
         jnp.zeros((N2, DT - 44), jnp.float32)], axis=1)
    idx3 = jnp.pad(topk.reshape(-1), (0, E_PAD - E)).reshape(NW, NCH, CH)
    gathered = _sc_gather(table, idx3)
    g3 = gathered.reshape(E_PAD // NN, NN, DT)
    return _tc_dense(
        features1, x1, nuv1.reshape(N1, 9), topk, g3,
        q1_w, q1_b.reshape(1, ND), q2_w, q2_b.reshape(1, NH * ND),
        g1_w, g1_b.reshape(1, ND), g2_w, g2_b.reshape(1, ND),
        k1_w, k2_w, v1_w, v2_w, d1_w, d2_w)


# speedup vs baseline: 1.9201x; 1.2454x over previous
import functools

import jax
import jax.numpy as jnp
from jax import lax
from jax.experimental import pallas as pl
from jax.experimental.pallas import tpu as pltpu
from jax.experimental.pallas import tpu_sc as plsc

N1 = 50000
N2 = 50000
NN = 16
NI = 128
ND = 32
NH = 4

DT = 48
NW = 32
CH = 128
NCH = 196
EPW = NCH * CH
E = N1 * NN
E_PAD = NW * EPW
B = 400


def _sc_gather(table, idx3):
    mesh = plsc.VectorSubcoreMesh(core_axis_name="c", subcore_axis_name="s")

    @functools.partial(
        pl.kernel,
        mesh=mesh,
        compiler_params=pltpu.CompilerParams(use_tc_tiling_on_sc=False),
        out_type=jax.ShapeDtypeStruct((E_PAD, DT), jnp.float32),
        scratch_types=[
            pltpu.VMEM((NCH, CH), jnp.int32),
            pltpu.VMEM((2, CH, DT), jnp.float32),
            pltpu.SemaphoreType.DMA,
            pltpu.SemaphoreType.DMA,
        ],
    )
    def k(table_hbm, idx_hbm, out_hbm, idx_v, rows_v, sem_a, sem_b):
        wid = lax.axis_index("s") * 2 + lax.axis_index("c")
        base = wid * EPW
        pltpu.sync_copy(idx_hbm.at[wid], idx_v)
        pltpu.async_copy(table_hbm.at[idx_v.at[0]], rows_v.at[0], sem_a)

        def body(jj, carry):
            j0 = 2 * jj
            pltpu.async_copy(table_hbm.at[idx_v.at[j0 + 1]], rows_v.at[1], sem_b)
            pltpu.make_async_copy(
                table_hbm.at[idx_v.at[j0]], rows_v.at[0], sem_a).wait()
            pltpu.sync_copy(rows_v.at[0], out_hbm.at[pl.ds(base + j0 * CH, CH)])

            @pl.when(j0 + 2 < NCH)
            def _():
                pltpu.async_copy(
                    table_hbm.at[idx_v.at[j0 + 2]], rows_v.at[0], sem_a)

            pltpu.make_async_copy(
                table_hbm.at[idx_v.at[j0 + 1]], rows_v.at[1], sem_b).wait()
            pltpu.sync_copy(
                rows_v.at[1], out_hbm.at[pl.ds(base + (j0 + 1) * CH, CH)])
            return carry

        lax.fori_loop(0, NCH // 2, body, 0)

    return k(table, idx3)


def _elu(x):
    xn = jnp.minimum(x, 0.0)
    e = jnp.exp(xn) - 1.0
    t = xn * (1.0 + xn * (0.5 + xn * (1.0 / 6.0 + xn * (1.0 / 24.0
        + xn * (1.0 / 120.0 + xn * (1.0 / 720.0))))))
    em1 = jnp.where(xn > -0.25, t, e)
    return jnp.where(x > 0, x, em1)


def _dot(a, b):
    return lax.dot_general(a, b, (((1,), (0,)), ((), ())),
                           preferred_element_type=jnp.float32)


def _dense_body(f1_ref, x1_ref, nuv_ref, tk_ref, g_ref,
                q1w_ref, q1b_ref, q2w_ref, q2b_ref, g1w_ref, g1b_ref,
                g2w_ref, g2b_ref, k1_ref, k2_ref, v1_ref, v2_ref,
                d1_ref, d2_ref, out_ref):
    h = _elu(_dot(f1_ref[...], q1w_ref[...]) + q1b_ref[...])
    Q = _dot(h, q2w_ref[...]) + q2b_ref[...]

    g = g_ref[...]
    nuv = nuv_ref[...]
    x1 = x1_ref[...]

    dx = [g[:, :, 32 + m] - x1[:, m][:, None] for m in range(3)]
    nv = [g[:, :, 35 + p] for p in range(9)]

    cols = []
    for kk in range(3):
        acc = nuv[:, 3 * kk][:, None] * dx[0]
        acc += nuv[:, 3 * kk + 1][:, None] * dx[1]
        acc += nuv[:, 3 * kk + 2][:, None] * dx[2]
        cols.append(acc)
    for r in range(3):
        for kk in range(3):
            acc = nuv[:, 3 * kk][:, None] * nv[3 * r]
            acc += nuv[:, 3 * kk + 1][:, None] * nv[3 * r + 1]
            acc += nuv[:, 3 * kk + 2][:, None] * nv[3 * r + 2]
            cols.append(acc)
    RL = jnp.stack(cols, axis=-1)

    d2 = dx[0] * dx[0] + dx[1] * dx[1] + dx[2] * dx[2]
    wgt = jnp.exp(-0.5 * d2)

    geo1 = _elu(_dot(RL.reshape(B * NN, 12), g1w_ref[...]) + g1b_ref[...])
    geo = _dot(geo1, g2w_ref[...]) + g2b_ref[...]
    geo3 = geo.reshape(B, NN, ND) * g[:, :, 0:32] * wgt[:, :, None]
    geo = geo3.reshape(B * NN, ND)

    K2 = _dot(_elu(_dot(geo, k1_ref[...])), k2_ref[...])
    V2 = _dot(_elu(_dot(geo, v1_ref[...])), v2_ref[...])
    K3 = K2.reshape(B, NN, ND)
    V3 = V2.reshape(B, NN, ND)

    mz = jnp.where(tk_ref[...] == 0, 0.0, 1.0)
    Km = K3 * mz[:, :, None]
    ones_v = jnp.ones((ND, 1), jnp.float32)
    outs = []
    for hh in range(NH):
        Qh = Q[:, ND * hh:ND * (hh + 1)]
        prod = (Km * Qh[:, None, :]).reshape(B * NN, ND)
        Mqh = _dot(prod, ones_v)
        wv = (Mqh * V2).reshape(B, NN, ND)
        outs.append(jnp.sum(wv, axis=1))
    o = jnp.concatenate(outs, axis=1)
    out_ref[...] = _elu(_dot(_elu(_dot(o, d1_ref[...])), d2_ref[...]))


def _row_spec(*shape):
    return pl.BlockSpec(shape, lambda b: (b,) + (0,) * (len(shape) - 1))


def _w_spec(*shape):
    return pl.BlockSpec(shape, lambda b: (0,) * len(shape))


def _tc_dense(f1, x1, nuv9, topk, g3, q1w, q1b, q2w, q2b, g1w, g1b,
              g2w, g2b, k1w, k2w, v1w, v2w, d1w, d2w):
    grid = (N1 // B,)
    in_specs = [
        _row_spec(B, NI),
        _row_spec(B, 3),
        _row_spec(B, 9),
        _row_spec(B, NN),
        _row_spec(B, NN, DT),
        _w_spec(NI, ND), _w_spec(1, ND), _w_spec(ND, NH * ND), _w_spec(1, NH * ND),
        _w_spec(12, ND), _w_spec(1, ND), _w_spec(ND, ND), _w_spec(1, ND),
        _w_spec(ND, ND), _w_spec(ND, ND), _w_spec(ND, ND), _w_spec(ND, ND),
        _w_spec(NH * ND, ND), _w_spec(ND, ND),
    ]
    return pl.pallas_call(
        _dense_body,
        grid=grid,
        in_specs=in_specs,
        out_specs=_row_spec(B, ND),
        out_shape=jax.ShapeDtypeStruct((N1, ND), jnp.float32),
    )(f1, x1, nuv9, topk, g3, q1w, q1b, q2w, q2b, g1w, g1b,
      g2w, g2b, k1w, k2w, v1w, v2w, d1w, d2w)


def kernel(features1, features2, x1, x2, nuv1, nuv2, topk,
           q1_w, q1_b, q2_w, q2_b, g1_w, g1_b, g2_w, g2_b,
           k1_w, k2_w, v1_w, v2_w, d1_w, d2_w):
    table = jnp.concatenate(
        [features2, x2, nuv2.reshape(N2, 9),
         jnp.zeros((N2, DT - 44), jnp.float32)], axis=1)
    idx3 = jnp.pad(topk.reshape(-1), (0, E_PAD - E)).reshape(NW, NCH, CH)
    gathered = _sc_gather(table, idx3)
    g3 = gathered.reshape(E_PAD // NN, NN, DT)
    return _tc_dense(
        features1, x1, nuv1.reshape(N1, 9), topk, g3,
        q1_w, q1_b.reshape(1, ND), q2_w, q2_b.reshape(1, NH * ND),
        g1_w, g1_b.reshape(1, ND), g2_w, g2_b.reshape(1, ND),
        k1_w, k2_w, v1_w, v2_w, d1_w, d2_w)


# speedup vs baseline: 8.1090x; 4.2233x over previous
import functools

import jax
import jax.numpy as jnp
import numpy as np
from jax import lax
from jax.experimental import pallas as pl
from jax.experimental.pallas import tpu as pltpu
from jax.experimental.pallas import tpu_sc as plsc

N1 = 50000
N2 = 50000
NN = 16
NI = 128
ND = 32
NH = 4

NW = 32
CH = 128
NCH = 196
EPW = NCH * CH
E = N1 * NN
E_PAD = NW * EPW
NNODE = E_PAD // NN
B = 400
B4 = B * 4


def _sc_gather_rl(ftab, gtab, ntab, idx3):
    mesh = plsc.VectorSubcoreMesh(core_axis_name="c", subcore_axis_name="s")

    def chunk_compute(gbuf, nbuf, rbuf):
        iota = lax.iota(jnp.int32, 16)
        for g in range(CH // 16):
            rows = iota + (g * 16)
            gfull = jnp.full((16,), g, jnp.int32)

            def ncol(c):
                return plsc.load_gather(nbuf, [gfull, jnp.full((16,), c, jnp.int32)])

            def gcol(c):
                return plsc.load_gather(gbuf, [rows, jnp.full((16,), c, jnp.int32)])

            dx = [gcol(m) - ncol(1 + m) for m in range(3)]
            nv = [gcol(3 + p) for p in range(9)]
            nuv = [ncol(4 + a) for a in range(9)]
            d2 = dx[0] * dx[0] + dx[1] * dx[1] + dx[2] * dx[2]
            wgt = jnp.exp(-0.5 * d2)
            for kk in range(3):
                rl = nuv[3 * kk] * dx[0] + nuv[3 * kk + 1] * dx[1] \
                    + nuv[3 * kk + 2] * dx[2]
                plsc.store_scatter(rbuf, [rows, jnp.full((16,), kk, jnp.int32)], rl)
            for r in range(3):
                for kk in range(3):
                    rl = nuv[3 * kk] * nv[3 * r] + nuv[3 * kk + 1] * nv[3 * r + 1] \
                        + nuv[3 * kk + 2] * nv[3 * r + 2]
                    plsc.store_scatter(
                        rbuf, [rows, jnp.full((16,), 3 + 3 * r + kk, jnp.int32)], rl)
            plsc.store_scatter(rbuf, [rows, jnp.full((16,), 12, jnp.int32)], wgt)
            zero = jnp.zeros((16,), jnp.float32)
            for c in (13, 14, 15):
                plsc.store_scatter(rbuf, [rows, jnp.full((16,), c, jnp.int32)], zero)

    @functools.partial(
        pl.kernel,
        mesh=mesh,
        compiler_params=pltpu.CompilerParams(
            use_tc_tiling_on_sc=False, needs_layout_passes=False),
        out_type=(
            jax.ShapeDtypeStruct((E_PAD, 32), jnp.float32),
            jax.ShapeDtypeStruct((E_PAD, 16), jnp.float32),
        ),
        scratch_types=[
            pltpu.VMEM((NCH, CH), jnp.int32),
            pltpu.VMEM((2, CH, 32), jnp.float32),
            pltpu.VMEM((CH, 16), jnp.float32),
            pltpu.VMEM((CH, 16), jnp.float32),
            pltpu.VMEM((CH // 16, 16), jnp.float32),
            pltpu.VMEM((CH // 16, 16), jnp.float32),
            pltpu.VMEM((CH, 16), jnp.float32),
            pltpu.SemaphoreType.DMA,
            pltpu.SemaphoreType.DMA,
            pltpu.SemaphoreType.DMA,
            pltpu.SemaphoreType.DMA,
            pltpu.SemaphoreType.DMA,
            pltpu.SemaphoreType.DMA,
        ],
    )
    def k(ftab_hbm, gtab_hbm, ntab_hbm, idx_hbm, outf_hbm, outr_hbm,
          idx_v, fbuf, gbuf_a, gbuf_b, nbuf_a, nbuf_b, rbuf,
          sf_a, sf_b, sg_a, sg_b, sn_a, sn_b):
        wid = lax.axis_index("s") * 2 + lax.axis_index("c")
        ebase = wid * EPW
        nbase = wid * (EPW // NN)
        pltpu.sync_copy(idx_hbm.at[wid], idx_v)

        def fire(j, fslot, gbuf, nbuf, sf, sg, sn):
            pltpu.async_copy(ftab_hbm.at[idx_v.at[j]], fbuf.at[fslot], sf)
            pltpu.async_copy(gtab_hbm.at[idx_v.at[j]], gbuf, sg)
            pltpu.async_copy(
                ntab_hbm.at[pl.ds(nbase + j * (CH // NN), CH // NN)],
                nbuf, sn)

        def finish(j, fslot, gbuf, nbuf, sf, sg, sn):
            pltpu.make_async_copy(
                ftab_hbm.at[idx_v.at[j]], fbuf.at[fslot], sf).wait()
            pltpu.make_async_copy(
                gtab_hbm.at[idx_v.at[j]], gbuf, sg).wait()
            pltpu.make_async_copy(
                ntab_hbm.at[pl.ds(nbase + j * (CH // NN), CH // NN)],
                nbuf, sn).wait()
            chunk_compute(gbuf, nbuf, rbuf)
            pltpu.sync_copy(
                fbuf.at[fslot], outf_hbm.at[pl.ds(ebase + j * CH, CH)])
            pltpu.sync_copy(
                rbuf, outr_hbm.at[pl.ds(ebase + j * CH, CH)])

        fire(0, 0, gbuf_a, nbuf_a, sf_a, sg_a, sn_a)

        def body(jj, carry):
            j0 = 2 * jj
            fire(j0 + 1, 1, gbuf_b, nbuf_b, sf_b, sg_b, sn_b)
            finish(j0, 0, gbuf_a, nbuf_a, sf_a, sg_a, sn_a)

            @pl.when(j0 + 2 < NCH)
            def _():
                fire(j0 + 2, 0, gbuf_a, nbuf_a, sf_a, sg_a, sn_a)

            finish(j0 + 1, 1, gbuf_b, nbuf_b, sf_b, sg_b, sn_b)
            return carry

        lax.fori_loop(0, NCH // 2, body, 0)

    return k(ftab, gtab, ntab, idx3)


def _elu(x):
    xn = jnp.minimum(x, 0.0)
    e = jnp.exp(xn) - 1.0
    t = xn * (1.0 + xn * (0.5 + xn * (1.0 / 6.0 + xn * (1.0 / 24.0
        + xn * (1.0 / 120.0 + xn * (1.0 / 720.0))))))
    em1 = jnp.where(xn > -0.25, t, e)
    return jnp.where(x > 0, x, em1)


def _dot(a, b):
    return lax.dot_general(a, b, (((1,), (0,)), ((), ())),
                           preferred_element_type=jnp.float32)


def _dense_body(f1_ref, fP_ref, rP_ref, tk4_ref,
                q1w_ref, q1b_ref, q2w_ref, q2b_ref,
                g1w4_ref, g1b4_ref, g2w4_ref, g2b4_ref,
                k14_ref, k24_ref, v14_ref, v24_ref,
                d1_ref, d2_ref,
                selw_ref, sel4_ref, tsel_ref, g32_ref, f0_ref,
                out_ref):
    h = _elu(_dot(f1_ref[...], q1w_ref[...]) + q1b_ref[...])
    Q = _dot(h, q2w_ref[...]) + q2b_ref[...]

    rP = rP_ref[...]
    fP = fP_ref[...]

    wgtP = _dot(rP, selw_ref[...])
    geo1 = _elu(_dot(rP, g1w4_ref[...]) + g1b4_ref[...])
    geoP = (_dot(geo1, g2w4_ref[...]) + g2b4_ref[...]) * fP * wgtP

    K2P = _dot(_elu(_dot(geoP, k14_ref[...])), k24_ref[...])
    VP = _dot(_elu(_dot(geoP, v14_ref[...])), v24_ref[...])

    mz4 = jnp.where(tk4_ref[...] == 0, 0.0, 1.0)
    K2m = K2P * _dot(mz4, sel4_ref[...])

    Qt = _dot(Q, tsel_ref[...])
    K2m3 = K2m.reshape(B, 4, 128)
    g32 = g32_ref[...]
    f0 = f0_ref[...]
    outs = []
    for hh in range(NH):
        Qth = Qt[:, 128 * hh:128 * (hh + 1)]
        prod = (K2m3 * Qth[:, None, :]).reshape(B4, 128)
        MqB = _dot(prod, g32)
        wv = MqB * VP
        s32 = _dot(wv, f0)
        outs.append(jnp.sum(s32.reshape(B, 4, ND), axis=1))
    o = jnp.concatenate(outs, axis=1)
    out_ref[...] = _elu(_dot(_elu(_dot(o, d1_ref[...])), d2_ref[...]))


def _row_spec(*shape):
    return pl.BlockSpec(shape, lambda b: (b,) + (0,) * (len(shape) - 1))


def _w_spec(*shape):
    return pl.BlockSpec(shape, lambda b: (0,) * len(shape))


def _tc_dense(f1, fP, rP, tk4, q1w, q1b, q2w, q2b, g1w4, g1b4, g2w4, g2b4,
              k14, k24, v14, v24, d1w, d2w, selw, sel4, tsel, g32, f0):
    grid = (N1 // B,)
    in_specs = [
        _row_spec(B, NI),
        _row_spec(B4, 128),
        _row_spec(B4, 64),
        _row_spec(B4, 4),
        _w_spec(NI, ND), _w_spec(1, ND), _w_spec(ND, NH * ND), _w_spec(1, NH * ND),
        _w_spec(64, 128), _w_spec(1, 128), _w_spec(128, 128), _w_spec(1, 128),
        _w_spec(128, 128), _w_spec(128, 128), _w_spec(128, 128), _w_spec(128, 128),
        _w_spec(NH * ND, ND), _w_spec(ND, ND),
        _w_spec(64, 128), _w_spec(4, 128), _w_spec(128, 512), _w_spec(128, 128),
        _w_spec(128, ND),
    ]
    return pl.pallas_call(
        _dense_body,
        grid=grid,
        in_specs=in_specs,
        out_specs=_row_spec(B, ND),
        out_shape=jax.ShapeDtypeStruct((N1, ND), jnp.float32),
    )(f1, fP, rP, tk4, q1w, q1b, q2w, q2b, g1w4, g1b4, g2w4, g2b4,
      k14, k24, v14, v24, d1w, d2w, selw, sel4, tsel, g32, f0)


def _blkdiag4(w):
    a, b = w.shape
    z = jnp.zeros((a, b), w.dtype)
    rows = []
    for i in range(4):
        rows.append(jnp.concatenate(
            [w if j == i else z for j in range(4)], axis=1))
    return jnp.concatenate(rows, axis=0)


def _np_const(arr):
    return jnp.asarray(arr, jnp.float32)


def kernel(features1, features2, x1, x2, nuv1, nuv2, topk,
           q1_w, q1_b, q2_w, q2_b, g1_w, g1_b, g2_w, g2_b,
           k1_w, k2_w, v1_w, v2_w, d1_w, d2_w):
    f32 = jnp.float32
    gtab = jnp.concatenate(
        [x2, nuv2.reshape(N2, 9), jnp.zeros((N2, 4), f32)], axis=1)
    ntab = jnp.concatenate(
        [jnp.zeros((N1, 1), f32), x1, nuv1.reshape(N1, 9),
         jnp.zeros((N1, 3), f32)], axis=1)
    ntab = jnp.pad(ntab, ((0, NNODE - N1), (0, 0)))
    idx3 = jnp.pad(topk.reshape(-1), (0, E_PAD - E)).reshape(NW, NCH, CH)
    fE, rE = _sc_gather_rl(features2, gtab, ntab, idx3)
    fP = fE.reshape(E_PAD // 4, 128)
    rP = rE.reshape(E_PAD // 4, 64)

    tk4 = jnp.pad(topk.reshape(E // 4, 4), ((0, E_PAD // 4 - E // 4), (0, 0)))

    g1p = jnp.concatenate([g1_w, jnp.zeros((4, ND), f32)], axis=0)
    g1w4 = _blkdiag4(g1p)
    g1b4 = jnp.tile(g1_b, 4).reshape(1, 128)
    g2w4 = _blkdiag4(g2_w)
    g2b4 = jnp.tile(g2_b, 4).reshape(1, 128)
    k14 = _blkdiag4(k1_w)
    k24 = _blkdiag4(k2_w)
    v14 = _blkdiag4(v1_w)
    v24 = _blkdiag4(v2_w)

    selw = np.zeros((64, 128), np.float32)
    for jl in range(4):
        selw[jl * 16 + 12, jl * 32:(jl + 1) * 32] = 1.0
    sel4 = np.zeros((4, 128), np.float32)
    for jl in range(4):
        sel4[jl, jl * 32:(jl + 1) * 32] = 1.0
    tsel = np.zeros((128, 512), np.float32)
    for hh in range(4):
        for jl in range(4):
            for d in range(ND):
                tsel[hh * 32 + d, hh * 128 + jl * 32 + d] = 1.0
    g32 = np.zeros((128, 128), np.float32)
    for jl in range(4):
        g32[jl * 32:(jl + 1) * 32, jl * 32:(jl + 1) * 32] = 1.0
    f0 = np.zeros((128, ND), np.float32)
    for jl in range(4):
        for d in range(ND):
            f0[jl * 32 + d, d] = 1.0

    return _tc_dense(
        features1, fP, rP, tk4,
        q1_w, q1_b.reshape(1, ND), q2_w, q2_b.reshape(1, NH * ND),
        g1w4, g1b4, g2w4, g2b4, k14, k24, v14, v24, d1_w, d2_w,
        _np_const(selw), _np_const(sel4), _np_const(tsel),
        _np_const(g32), _np_const(f0))


# speedup vs baseline: 8.3251x; 1.0266x over previous
import functools

import jax
import jax.numpy as jnp
import numpy as np
from jax import lax
from jax.experimental import pallas as pl
from jax.experimental.pallas import tpu as pltpu
from jax.experimental.pallas import tpu_sc as plsc

N1 = 50000
N2 = 50000
NN = 16
NI = 128
ND = 32
NH = 4

NW = 32
CH = 128
NCH = 196
EPW = NCH * CH
E = N1 * NN
E_PAD = NW * EPW
NNODE = E_PAD // NN
B = 800
B4 = B * 4


def _sc_gather_rl(ftab, gtab, ntab, idx3):
    mesh = plsc.VectorSubcoreMesh(core_axis_name="c", subcore_axis_name="s")

    def chunk_compute(gbuf, nbuf, rbuf):
        iota = lax.iota(jnp.int32, 16)
        for g in range(CH // 16):
            rows = iota + (g * 16)
            gfull = jnp.full((16,), g, jnp.int32)

            def ncol(c):
                return plsc.load_gather(nbuf, [gfull, jnp.full((16,), c, jnp.int32)])

            def gcol(c):
                return plsc.load_gather(gbuf, [rows, jnp.full((16,), c, jnp.int32)])

            dx = [gcol(m) - ncol(1 + m) for m in range(3)]
            nv = [gcol(3 + p) for p in range(9)]
            nuv = [ncol(4 + a) for a in range(9)]
            d2 = dx[0] * dx[0] + dx[1] * dx[1] + dx[2] * dx[2]
            wgt = jnp.exp(-0.5 * d2)
            for kk in range(3):
                rl = nuv[3 * kk] * dx[0] + nuv[3 * kk + 1] * dx[1] \
                    + nuv[3 * kk + 2] * dx[2]
                plsc.store_scatter(rbuf, [rows, jnp.full((16,), kk, jnp.int32)], rl)
            for r in range(3):
                for kk in range(3):
                    rl = nuv[3 * kk] * nv[3 * r] + nuv[3 * kk + 1] * nv[3 * r + 1] \
                        + nuv[3 * kk + 2] * nv[3 * r + 2]
                    plsc.store_scatter(
                        rbuf, [rows, jnp.full((16,), 3 + 3 * r + kk, jnp.int32)], rl)
            plsc.store_scatter(rbuf, [rows, jnp.full((16,), 12, jnp.int32)], wgt)
            zero = jnp.zeros((16,), jnp.float32)
            for c in (13, 14, 15):
                plsc.store_scatter(rbuf, [rows, jnp.full((16,), c, jnp.int32)], zero)

    @functools.partial(
        pl.kernel,
        mesh=mesh,
        compiler_params=pltpu.CompilerParams(
            use_tc_tiling_on_sc=False, needs_layout_passes=False),
        out_type=(
            jax.ShapeDtypeStruct((E_PAD, 32), jnp.float32),
            jax.ShapeDtypeStruct((E_PAD, 16), jnp.float32),
        ),
        scratch_types=[
            pltpu.VMEM((NCH, CH), jnp.int32),
            pltpu.VMEM((2, CH, 32), jnp.float32),
            pltpu.VMEM((CH, 16), jnp.float32),
            pltpu.VMEM((CH, 16), jnp.float32),
            pltpu.VMEM((CH // 16, 16), jnp.float32),
            pltpu.VMEM((CH // 16, 16), jnp.float32),
            pltpu.VMEM((CH, 16), jnp.float32),
            pltpu.SemaphoreType.DMA,
            pltpu.SemaphoreType.DMA,
            pltpu.SemaphoreType.DMA,
            pltpu.SemaphoreType.DMA,
            pltpu.SemaphoreType.DMA,
            pltpu.SemaphoreType.DMA,
        ],
    )
    def k(ftab_hbm, gtab_hbm, ntab_hbm, idx_hbm, outf_hbm, outr_hbm,
          idx_v, fbuf, gbuf_a, gbuf_b, nbuf_a, nbuf_b, rbuf,
          sf_a, sf_b, sg_a, sg_b, sn_a, sn_b):
        wid = lax.axis_index("s") * 2 + lax.axis_index("c")
        ebase = wid * EPW
        nbase = wid * (EPW // NN)
        pltpu.sync_copy(idx_hbm.at[wid], idx_v)

        def fire(j, fslot, gbuf, nbuf, sf, sg, sn):
            pltpu.async_copy(ftab_hbm.at[idx_v.at[j]], fbuf.at[fslot], sf)
            pltpu.async_copy(gtab_hbm.at[idx_v.at[j]], gbuf, sg)
            pltpu.async_copy(
                ntab_hbm.at[pl.ds(nbase + j * (CH // NN), CH // NN)],
                nbuf, sn)

        def finish(j, fslot, gbuf, nbuf, sf, sg, sn):
            pltpu.make_async_copy(
                ftab_hbm.at[idx_v.at[j]], fbuf.at[fslot], sf).wait()
            pltpu.make_async_copy(
                gtab_hbm.at[idx_v.at[j]], gbuf, sg).wait()
            pltpu.make_async_copy(
                ntab_hbm.at[pl.ds(nbase + j * (CH // NN), CH // NN)],
                nbuf, sn).wait()
            chunk_compute(gbuf, nbuf, rbuf)
            pltpu.sync_copy(
                fbuf.at[fslot], outf_hbm.at[pl.ds(ebase + j * CH, CH)])
            pltpu.sync_copy(
                rbuf, outr_hbm.at[pl.ds(ebase + j * CH, CH)])

        fire(0, 0, gbuf_a, nbuf_a, sf_a, sg_a, sn_a)

        def body(jj, carry):
            j0 = 2 * jj
            fire(j0 + 1, 1, gbuf_b, nbuf_b, sf_b, sg_b, sn_b)
            finish(j0, 0, gbuf_a, nbuf_a, sf_a, sg_a, sn_a)

            @pl.when(j0 + 2 < NCH)
            def _():
                fire(j0 + 2, 0, gbuf_a, nbuf_a, sf_a, sg_a, sn_a)

            finish(j0 + 1, 1, gbuf_b, nbuf_b, sf_b, sg_b, sn_b)
            return carry

        lax.fori_loop(0, NCH // 2, body, 0)

    return k(ftab, gtab, ntab, idx3)


def _elu(x):
    xn = jnp.minimum(x, 0.0)
    e = jnp.exp(xn) - 1.0
    t = xn * (1.0 + xn * (0.5 + xn * (1.0 / 6.0 + xn * (1.0 / 24.0
        + xn * (1.0 / 120.0 + xn * (1.0 / 720.0))))))
    em1 = jnp.where(xn > -0.25, t, e)
    return jnp.where(x > 0, x, em1)


def _dot(a, b):
    return lax.dot_general(a, b, (((1,), (0,)), ((), ())),
                           preferred_element_type=jnp.float32)


def _dense_body(f1_ref, fP_ref, rP_ref, tk4_ref,
                q1w_ref, q1b_ref, q2w_ref, q2b_ref,
                g1w4_ref, g1b4_ref, g2w4_ref, g2b4_ref,
                k14_ref, k24_ref, v14_ref, v24_ref,
                d1_ref, d2_ref,
                selw_ref, sel4_ref, tsel_ref, g32_ref, f0_ref,
                out_ref):
    h = _elu(_dot(f1_ref[...], q1w_ref[...]) + q1b_ref[...])
    Q = _dot(h, q2w_ref[...]) + q2b_ref[...]

    rP = rP_ref[...]
    fP = fP_ref[...]

    wgtP = _dot(rP, selw_ref[...])
    geo1 = _elu(_dot(rP, g1w4_ref[...]) + g1b4_ref[...])
    geoP = (_dot(geo1, g2w4_ref[...]) + g2b4_ref[...]) * fP * wgtP

    K2P = _dot(_elu(_dot(geoP, k14_ref[...])), k24_ref[...])
    VP = _dot(_elu(_dot(geoP, v14_ref[...])), v24_ref[...])

    mz4 = jnp.where(tk4_ref[...] == 0, 0.0, 1.0)
    K2m = K2P * _dot(mz4, sel4_ref[...])

    Qt = _dot(Q, tsel_ref[...])
    K2m3 = K2m.reshape(B, 4, 128)
    g32 = g32_ref[...]
    f0 = f0_ref[...]
    outs = []
    for hh in range(NH):
        Qth = Qt[:, 128 * hh:128 * (hh + 1)]
        prod = (K2m3 * Qth[:, None, :]).reshape(B4, 128)
        MqB = _dot(prod, g32)
        wv = MqB * VP
        s32 = _dot(wv, f0)
        outs.append(jnp.sum(s32.reshape(B, 4, ND), axis=1))
    o = jnp.concatenate(outs, axis=1)
    out_ref[...] = _elu(_dot(_elu(_dot(o, d1_ref[...])), d2_ref[...]))


def _row_spec(*shape):
    return pl.BlockSpec(shape, lambda b: (b,) + (0,) * (len(shape) - 1))


def _w_spec(*shape):
    return pl.BlockSpec(shape, lambda b: (0,) * len(shape))


def _tc_dense(f1, fP, rP, tk4, q1w, q1b, q2w, q2b, g1w4, g1b4, g2w4, g2b4,
              k14, k24, v14, v24, d1w, d2w, selw, sel4, tsel, g32, f0):
    grid = (N1 // B,)
    in_specs = [
        _row_spec(B, NI),
        _row_spec(B4, 128),
        _row_spec(B4, 64),
        _row_spec(B4, 4),
        _w_spec(NI, ND), _w_spec(1, ND), _w_spec(ND, NH * ND), _w_spec(1, NH * ND),
        _w_spec(64, 128), _w_spec(1, 128), _w_spec(128, 128), _w_spec(1, 128),
        _w_spec(128, 128), _w_spec(128, 128), _w_spec(128, 128), _w_spec(128, 128),
        _w_spec(NH * ND, ND), _w_spec(ND, ND),
        _w_spec(64, 128), _w_spec(4, 128), _w_spec(128, 512), _w_spec(128, 128),
        _w_spec(128, ND),
    ]
    return pl.pallas_call(
        _dense_body,
        grid=grid,
        in_specs=in_specs,
        out_specs=_row_spec(B, ND),
        out_shape=jax.ShapeDtypeStruct((N1, ND), jnp.float32),
    )(f1, fP, rP, tk4, q1w, q1b, q2w, q2b, g1w4, g1b4, g2w4, g2b4,
      k14, k24, v14, v24, d1w, d2w, selw, sel4, tsel, g32, f0)


def _blkdiag4(w):
    a, b = w.shape
    z = jnp.zeros((a, b), w.dtype)
    rows = []
    for i in range(4):
        rows.append(jnp.concatenate(
            [w if j == i else z for j in range(4)], axis=1))
    return jnp.concatenate(rows, axis=0)


def _np_const(arr):
    return jnp.asarray(arr, jnp.float32)


def kernel(features1, features2, x1, x2, nuv1, nuv2, topk,
           q1_w, q1_b, q2_w, q2_b, g1_w, g1_b, g2_w, g2_b,
           k1_w, k2_w, v1_w, v2_w, d1_w, d2_w):
    f32 = jnp.float32
    gtab = jnp.concatenate(
        [x2, nuv2.reshape(N2, 9), jnp.zeros((N2, 4), f32)], axis=1)
    ntab = jnp.concatenate(
        [jnp.zeros((N1, 1), f32), x1, nuv1.reshape(N1, 9),
         jnp.zeros((N1, 3), f32)], axis=1)
    ntab = jnp.pad(ntab, ((0, NNODE - N1), (0, 0)))
    idx3 = jnp.pad(topk.reshape(-1), (0, E_PAD - E)).reshape(NW, NCH, CH)
    fE, rE = _sc_gather_rl(features2, gtab, ntab, idx3)
    fP = fE.reshape(E_PAD // 4, 128)
    rP = rE.reshape(E_PAD // 4, 64)

    tk4 = jnp.pad(topk.reshape(E // 4, 4), ((0, E_PAD // 4 - E // 4), (0, 0)))

    g1p = jnp.concatenate([g1_w, jnp.zeros((4, ND), f32)], axis=0)
    g1w4 = _blkdiag4(g1p)
    g1b4 = jnp.tile(g1_b, 4).reshape(1, 128)
    g2w4 = _blkdiag4(g2_w)
    g2b4 = jnp.tile(g2_b, 4).reshape(1, 128)
    k14 = _blkdiag4(k1_w)
    k24 = _blkdiag4(k2_w)
    v14 = _blkdiag4(v1_w)
    v24 = _blkdiag4(v2_w)

    selw = np.zeros((64, 128), np.float32)
    for jl in range(4):
        selw[jl * 16 + 12, jl * 32:(jl + 1) * 32] = 1.0
    sel4 = np.zeros((4, 128), np.float32)
    for jl in range(4):
        sel4[jl, jl * 32:(jl + 1) * 32] = 1.0
    tsel = np.zeros((128, 512), np.float32)
    for hh in range(4):
        for jl in range(4):
            for d in range(ND):
                tsel[hh * 32 + d, hh * 128 + jl * 32 + d] = 1.0
    g32 = np.zeros((128, 128), np.float32)
    for jl in range(4):
        g32[jl * 32:(jl + 1) * 32, jl * 32:(jl + 1) * 32] = 1.0
    f0 = np.zeros((128, ND), np.float32)
    for jl in range(4):
        for d in range(ND):
            f0[jl * 32 + d, d] = 1.0

    return _tc_dense(
        features1, fP, rP, tk4,
        q1_w, q1_b.reshape(1, ND), q2_w, q2_b.reshape(1, NH * ND),
        g1w4, g1b4, g2w4, g2b4, k14, k24, v14, v24, d1_w, d2_w,
        _np_const(selw), _np_const(sel4), _np_const(tsel),
        _np_const(g32), _np_const(f0))


# speedup vs baseline: 8.3626x; 1.0045x over previous
import functools

import jax
import jax.numpy as jnp
import numpy as np
from jax import lax
from jax.experimental import pallas as pl
from jax.experimental.pallas import tpu as pltpu
from jax.experimental.pallas import tpu_sc as plsc

N1 = 50000
N2 = 50000
NN = 16
NI = 128
ND = 32
NH = 4

NW = 32
CH = 128
NCH = 196
EPW = NCH * CH
E = N1 * NN
E_PAD = NW * EPW
NNODE = E_PAD // NN
B = 1000
B4 = B * 4


def _sc_gather_rl(ftab, gtab, ntab, idx3):
    mesh = plsc.VectorSubcoreMesh(core_axis_name="c", subcore_axis_name="s")

    def chunk_compute(gbuf, nbuf, rbuf):
        iota = lax.iota(jnp.int32, 16)
        for g in range(CH // 16):
            rows = iota + (g * 16)
            gfull = jnp.full((16,), g, jnp.int32)

            def ncol(c):
                return plsc.load_gather(nbuf, [gfull, jnp.full((16,), c, jnp.int32)])

            def gcol(c):
                return plsc.load_gather(gbuf, [rows, jnp.full((16,), c, jnp.int32)])

            dx = [gcol(m) - ncol(1 + m) for m in range(3)]
            nv = [gcol(3 + p) for p in range(9)]
            nuv = [ncol(4 + a) for a in range(9)]
            d2 = dx[0] * dx[0] + dx[1] * dx[1] + dx[2] * dx[2]
            wgt = jnp.exp(-0.5 * d2)
            for kk in range(3):
                rl = nuv[3 * kk] * dx[0] + nuv[3 * kk + 1] * dx[1] \
                    + nuv[3 * kk + 2] * dx[2]
                plsc.store_scatter(rbuf, [rows, jnp.full((16,), kk, jnp.int32)], rl)
            for r in range(3):
                for kk in range(3):
                    rl = nuv[3 * kk] * nv[3 * r] + nuv[3 * kk + 1] * nv[3 * r + 1] \
                        + nuv[3 * kk + 2] * nv[3 * r + 2]
                    plsc.store_scatter(
                        rbuf, [rows, jnp.full((16,), 3 + 3 * r + kk, jnp.int32)], rl)
            plsc.store_scatter(rbuf, [rows, jnp.full((16,), 12, jnp.int32)], wgt)
            zero = jnp.zeros((16,), jnp.float32)
            for c in (13, 14, 15):
                plsc.store_scatter(rbuf, [rows, jnp.full((16,), c, jnp.int32)], zero)

    @functools.partial(
        pl.kernel,
        mesh=mesh,
        compiler_params=pltpu.CompilerParams(
            use_tc_tiling_on_sc=False, needs_layout_passes=False),
        out_type=(
            jax.ShapeDtypeStruct((E_PAD, 32), jnp.float32),
            jax.ShapeDtypeStruct((E_PAD, 16), jnp.float32),
        ),
        scratch_types=[
            pltpu.VMEM((NCH, CH), jnp.int32),
            pltpu.VMEM((2, CH, 32), jnp.float32),
            pltpu.VMEM((CH, 16), jnp.float32),
            pltpu.VMEM((CH, 16), jnp.float32),
            pltpu.VMEM((CH // 16, 16), jnp.float32),
            pltpu.VMEM((CH // 16, 16), jnp.float32),
            pltpu.VMEM((CH, 16), jnp.float32),
            pltpu.SemaphoreType.DMA,
            pltpu.SemaphoreType.DMA,
            pltpu.SemaphoreType.DMA,
            pltpu.SemaphoreType.DMA,
            pltpu.SemaphoreType.DMA,
            pltpu.SemaphoreType.DMA,
        ],
    )
    def k(ftab_hbm, gtab_hbm, ntab_hbm, idx_hbm, outf_hbm, outr_hbm,
          idx_v, fbuf, gbuf_a, gbuf_b, nbuf_a, nbuf_b, rbuf,
          sf_a, sf_b, sg_a, sg_b, sn_a, sn_b):
        wid = lax.axis_index("s") * 2 + lax.axis_index("c")
        ebase = wid * EPW
        nbase = wid * (EPW // NN)
        pltpu.sync_copy(idx_hbm.at[wid], idx_v)

        def fire(j, fslot, gbuf, nbuf, sf, sg, sn):
            pltpu.async_copy(ftab_hbm.at[idx_v.at[j]], fbuf.at[fslot], sf)
            pltpu.async_copy(gtab_hbm.at[idx_v.at[j]], gbuf, sg)
            pltpu.async_copy(
                ntab_hbm.at[pl.ds(nbase + j * (CH // NN), CH // NN)],
                nbuf, sn)

        def finish(j, fslot, gbuf, nbuf, sf, sg, sn):
            pltpu.make_async_copy(
                ftab_hbm.at[idx_v.at[j]], fbuf.at[fslot], sf).wait()
            pltpu.make_async_copy(
                gtab_hbm.at[idx_v.at[j]], gbuf, sg).wait()
            pltpu.make_async_copy(
                ntab_hbm.at[pl.ds(nbase + j * (CH // NN), CH // NN)],
                nbuf, sn).wait()
            chunk_compute(gbuf, nbuf, rbuf)
            pltpu.sync_copy(
                fbuf.at[fslot], outf_hbm.at[pl.ds(ebase + j * CH, CH)])
            pltpu.sync_copy(
                rbuf, outr_hbm.at[pl.ds(ebase + j * CH, CH)])

        fire(0, 0, gbuf_a, nbuf_a, sf_a, sg_a, sn_a)

        def body(jj, carry):
            j0 = 2 * jj
            fire(j0 + 1, 1, gbuf_b, nbuf_b, sf_b, sg_b, sn_b)
            finish(j0, 0, gbuf_a, nbuf_a, sf_a, sg_a, sn_a)

            @pl.when(j0 + 2 < NCH)
            def _():
                fire(j0 + 2, 0, gbuf_a, nbuf_a, sf_a, sg_a, sn_a)

            finish(j0 + 1, 1, gbuf_b, nbuf_b, sf_b, sg_b, sn_b)
            return carry

        lax.fori_loop(0, NCH // 2, body, 0)

    return k(ftab, gtab, ntab, idx3)


def _elu(x):
    xn = jnp.minimum(x, 0.0)
    e = jnp.exp(xn) - 1.0
    t = xn * (1.0 + xn * (0.5 + xn * (1.0 / 6.0 + xn * (1.0 / 24.0
        + xn * (1.0 / 120.0 + xn * (1.0 / 720.0))))))
    em1 = jnp.where(xn > -0.25, t, e)
    return jnp.where(x > 0, x, em1)


def _dot(a, b):
    return lax.dot_general(a, b, (((1,), (0,)), ((), ())),
                           preferred_element_type=jnp.float32)


def _dense_body(f1_ref, fP_ref, rP_ref, tk4_ref,
                q1w_ref, q1b_ref, q2w_ref, q2b_ref,
                g1w4_ref, g1b4_ref, g2w4_ref, g2b4_ref,
                k14_ref, k24_ref, v14_ref, v24_ref,
                d1_ref, d2_ref,
                selw_ref, sel4_ref, tsel_ref, g32_ref, f0_ref,
                out_ref):
    h = _elu(_dot(f1_ref[...], q1w_ref[...]) + q1b_ref[...])
    Q = _dot(h, q2w_ref[...]) + q2b_ref[...]

    rP = rP_ref[...]
    fP = fP_ref[...]

    wgtP = _dot(rP, selw_ref[...])
    geo1 = _elu(_dot(rP, g1w4_ref[...]) + g1b4_ref[...])
    geoP = (_dot(geo1, g2w4_ref[...]) + g2b4_ref[...]) * fP * wgtP

    K2P = _dot(_elu(_dot(geoP, k14_ref[...])), k24_ref[...])
    VP = _dot(_elu(_dot(geoP, v14_ref[...])), v24_ref[...])

    mz4 = jnp.where(tk4_ref[...] == 0, 0.0, 1.0)
    K2m = K2P * _dot(mz4, sel4_ref[...])

    Qt = _dot(Q, tsel_ref[...])
    K2m3 = K2m.reshape(B, 4, 128)
    g32 = g32_ref[...]
    f0 = f0_ref[...]
    outs = []
    for hh in range(NH):
        Qth = Qt[:, 128 * hh:128 * (hh + 1)]
        prod = (K2m3 * Qth[:, None, :]).reshape(B4, 128)
        MqB = _dot(prod, g32)
        wv = MqB * VP
        s32 = _dot(wv, f0)
        outs.append(jnp.sum(s32.reshape(B, 4, ND), axis=1))
    o = jnp.concatenate(outs, axis=1)
    out_ref[...] = _elu(_dot(_elu(_dot(o, d1_ref[...])), d2_ref[...]))


def _row_spec(*shape):
    return pl.BlockSpec(shape, lambda b: (b,) + (0,) * (len(shape) - 1))


def _w_spec(*shape):
    return pl.BlockSpec(shape, lambda b: (0,) * len(shape))


def _tc_dense(f1, fP, rP, tk4, q1w, q1b, q2w, q2b, g1w4, g1b4, g2w4, g2b4,
              k14, k24, v14, v24, d1w, d2w, selw, sel4, tsel, g32, f0):
    grid = (N1 // B,)
    in_specs = [
        _row_spec(B, NI),
        _row_spec(B4, 128),
        _row_spec(B4, 64),
        _row_spec(B4, 4),
        _w_spec(NI, ND), _w_spec(1, ND), _w_spec(ND, NH * ND), _w_spec(1, NH * ND),
        _w_spec(64, 128), _w_spec(1, 128), _w_spec(128, 128), _w_spec(1, 128),
        _w_spec(128, 128), _w_spec(128, 128), _w_spec(128, 128), _w_spec(128, 128),
        _w_spec(NH * ND, ND), _w_spec(ND, ND),
        _w_spec(64, 128), _w_spec(4, 128), _w_spec(128, 512), _w_spec(128, 128),
        _w_spec(128, ND),
    ]
    return pl.pallas_call(
        _dense_body,
        grid=grid,
        in_specs=in_specs,
        out_specs=_row_spec(B, ND),
        out_shape=jax.ShapeDtypeStruct((N1, ND), jnp.float32),
    )(f1, fP, rP, tk4, q1w, q1b, q2w, q2b, g1w4, g1b4, g2w4, g2b4,
      k14, k24, v14, v24, d1w, d2w, selw, sel4, tsel, g32, f0)


def _blkdiag4(w):
    a, b = w.shape
    z = jnp.zeros((a, b), w.dtype)
    rows = []
    for i in range(4):
        rows.append(jnp.concatenate(
            [w if j == i else z for j in range(4)], axis=1))
    return jnp.concatenate(rows, axis=0)


def _np_const(arr):
    return jnp.asarray(arr, jnp.float32)


def kernel(features1, features2, x1, x2, nuv1, nuv2, topk,
           q1_w, q1_b, q2_w, q2_b, g1_w, g1_b, g2_w, g2_b,
           k1_w, k2_w, v1_w, v2_w, d1_w, d2_w):
    f32 = jnp.float32
    gtab = jnp.concatenate(
        [x2, nuv2.reshape(N2, 9), jnp.zeros((N2, 4), f32)], axis=1)
    ntab = jnp.concatenate(
        [jnp.zeros((N1, 1), f32), x1, nuv1.reshape(N1, 9),
         jnp.zeros((N1, 3), f32)], axis=1)
    ntab = jnp.pad(ntab, ((0, NNODE - N1), (0, 0)))
    idx3 = jnp.pad(topk.reshape(-1), (0, E_PAD - E)).reshape(NW, NCH, CH)
    fE, rE = _sc_gather_rl(features2, gtab, ntab, idx3)
    fP = fE.reshape(E_PAD // 4, 128)
    rP = rE.reshape(E_PAD // 4, 64)

    tk4 = jnp.pad(topk.reshape(E // 4, 4), ((0, E_PAD // 4 - E // 4), (0, 0)))

    g1p = jnp.concatenate([g1_w, jnp.zeros((4, ND), f32)], axis=0)
    g1w4 = _blkdiag4(g1p)
    g1b4 = jnp.tile(g1_b, 4).reshape(1, 128)
    g2w4 = _blkdiag4(g2_w)
    g2b4 = jnp.tile(g2_b, 4).reshape(1, 128)
    k14 = _blkdiag4(k1_w)
    k24 = _blkdiag4(k2_w)
    v14 = _blkdiag4(v1_w)
    v24 = _blkdiag4(v2_w)

    selw = np.zeros((64, 128), np.float32)
    for jl in range(4):
        selw[jl * 16 + 12, jl * 32:(jl + 1) * 32] = 1.0
    sel4 = np.zeros((4, 128), np.float32)
    for jl in range(4):
        sel4[jl, jl * 32:(jl + 1) * 32] = 1.0
    tsel = np.zeros((128, 512), np.float32)
    for hh in range(4):
        for jl in range(4):
            for d in range(ND):
                tsel[hh * 32 + d, hh * 128 + jl * 32 + d] = 1.0
    g32 = np.zeros((128, 128), np.float32)
    for jl in range(4):
        g32[jl * 32:(jl + 1) * 32, jl * 32:(jl + 1) * 32] = 1.0
    f0 = np.zeros((128, ND), np.float32)
    for jl in range(4):
        for d in range(ND):
            f0[jl * 32 + d, d] = 1.0

    return _tc_dense(
        features1, fP, rP, tk4,
        q1_w, q1_b.reshape(1, ND), q2_w, q2_b.reshape(1, NH * ND),
        g1w4, g1b4, g2w4, g2b4, k14, k24, v14, v24, d1_w, d2_w,
        _np_const(selw), _np_const(sel4), _np_const(tsel),
        _np_const(g32), _np_const(f0))


# speedup vs baseline: 9.2307x; 1.1038x over previous
import functools

import jax
import jax.numpy as jnp
import numpy as np
from jax import lax
from jax.experimental import pallas as pl
from jax.experimental.pallas import tpu as pltpu
from jax.experimental.pallas import tpu_sc as plsc

N1 = 50000
N2 = 50000
NN = 16
NI = 128
ND = 32
NH = 4

NW = 32
CH = 128
NCH = 196
EPW = NCH * CH
E = N1 * NN
E_PAD = NW * EPW
NNODE = E_PAD // NN
B = 1000
B4 = B * 4


def _sc_gather_rl(ftab, gtab, ntab, idx3):
    mesh = plsc.VectorSubcoreMesh(core_axis_name="c", subcore_axis_name="s")

    def chunk_compute(gbuf, nbuf, rbuf):
        iota = lax.iota(jnp.int32, 16)
        for g in range(CH // 16):
            rows = iota + (g * 16)
            gfull = jnp.full((16,), g, jnp.int32)

            def ncol(c):
                return plsc.load_gather(nbuf, [gfull, jnp.full((16,), c, jnp.int32)])

            def gcol(c):
                return plsc.load_gather(gbuf, [rows, jnp.full((16,), c, jnp.int32)])

            dx = [gcol(m) - ncol(1 + m) for m in range(3)]
            nv = [gcol(3 + p) for p in range(9)]
            nuv = [ncol(4 + a) for a in range(9)]
            d2 = dx[0] * dx[0] + dx[1] * dx[1] + dx[2] * dx[2]
            wgt = jnp.exp(-0.5 * d2)
            for kk in range(3):
                rl = nuv[3 * kk] * dx[0] + nuv[3 * kk + 1] * dx[1] \
                    + nuv[3 * kk + 2] * dx[2]
                plsc.store_scatter(rbuf, [rows, jnp.full((16,), kk, jnp.int32)], rl)
            for r in range(3):
                for kk in range(3):
                    rl = nuv[3 * kk] * nv[3 * r] + nuv[3 * kk + 1] * nv[3 * r + 1] \
                        + nuv[3 * kk + 2] * nv[3 * r + 2]
                    plsc.store_scatter(
                        rbuf, [rows, jnp.full((16,), 3 + 3 * r + kk, jnp.int32)], rl)
            plsc.store_scatter(rbuf, [rows, jnp.full((16,), 12, jnp.int32)], wgt)
            zero = jnp.zeros((16,), jnp.float32)
            for c in (13, 14, 15):
                plsc.store_scatter(rbuf, [rows, jnp.full((16,), c, jnp.int32)], zero)

    @functools.partial(
        pl.kernel,
        mesh=mesh,
        compiler_params=pltpu.CompilerParams(
            use_tc_tiling_on_sc=False, needs_layout_passes=False),
        out_type=(
            jax.ShapeDtypeStruct((E_PAD, 32), jnp.float32),
            jax.ShapeDtypeStruct((E_PAD, 16), jnp.float32),
        ),
        scratch_types=[
            pltpu.VMEM((NCH, CH), jnp.int32),
            pltpu.VMEM((2, CH, 32), jnp.float32),
            pltpu.VMEM((CH, 16), jnp.float32),
            pltpu.VMEM((CH, 16), jnp.float32),
            pltpu.VMEM((CH // 16, 16), jnp.float32),
            pltpu.VMEM((CH // 16, 16), jnp.float32),
            pltpu.VMEM((CH, 16), jnp.float32),
            pltpu.SemaphoreType.DMA,
            pltpu.SemaphoreType.DMA,
            pltpu.SemaphoreType.DMA,
            pltpu.SemaphoreType.DMA,
            pltpu.SemaphoreType.DMA,
            pltpu.SemaphoreType.DMA,
        ],
    )
    def k(ftab_hbm, gtab_hbm, ntab_hbm, idx_hbm, outf_hbm, outr_hbm,
          idx_v, fbuf, gbuf_a, gbuf_b, nbuf_a, nbuf_b, rbuf,
          sf_a, sf_b, sg_a, sg_b, sn_a, sn_b):
        wid = lax.axis_index("s") * 2 + lax.axis_index("c")
        ebase = wid * EPW
        nbase = wid * (EPW // NN)
        pltpu.sync_copy(idx_hbm.at[wid], idx_v)

        def fire(j, fslot, gbuf, nbuf, sf, sg, sn):
            pltpu.async_copy(ftab_hbm.at[idx_v.at[j]], fbuf.at[fslot], sf)
            pltpu.async_copy(gtab_hbm.at[idx_v.at[j]], gbuf, sg)
            pltpu.async_copy(
                ntab_hbm.at[pl.ds(nbase + j * (CH // NN), CH // NN)],
                nbuf, sn)

        def finish(j, fslot, gbuf, nbuf, sf, sg, sn):
            pltpu.make_async_copy(
                ftab_hbm.at[idx_v.at[j]], fbuf.at[fslot], sf).wait()
            pltpu.make_async_copy(
                gtab_hbm.at[idx_v.at[j]], gbuf, sg).wait()
            pltpu.make_async_copy(
                ntab_hbm.at[pl.ds(nbase + j * (CH // NN), CH // NN)],
                nbuf, sn).wait()
            chunk_compute(gbuf, nbuf, rbuf)
            pltpu.sync_copy(
                fbuf.at[fslot], outf_hbm.at[pl.ds(ebase + j * CH, CH)])
            pltpu.sync_copy(
                rbuf, outr_hbm.at[pl.ds(ebase + j * CH, CH)])

        fire(0, 0, gbuf_a, nbuf_a, sf_a, sg_a, sn_a)

        def body(jj, carry):
            j0 = 2 * jj
            fire(j0 + 1, 1, gbuf_b, nbuf_b, sf_b, sg_b, sn_b)
            finish(j0, 0, gbuf_a, nbuf_a, sf_a, sg_a, sn_a)

            @pl.when(j0 + 2 < NCH)
            def _():
                fire(j0 + 2, 0, gbuf_a, nbuf_a, sf_a, sg_a, sn_a)

            finish(j0 + 1, 1, gbuf_b, nbuf_b, sf_b, sg_b, sn_b)
            return carry

        lax.fori_loop(0, NCH // 2, body, 0)

    return k(ftab, gtab, ntab, idx3)


def _elu(x):
    xn = jnp.minimum(x, 0.0)
    e = jnp.exp(xn) - 1.0
    t = xn * (1.0 + xn * (0.5 + xn * (1.0 / 6.0)))
    em1 = jnp.where(xn > -0.0625, t, e)
    return jnp.where(x > 0, x, em1)


def _dot(a, b):
    return lax.dot_general(a, b, (((1,), (0,)), ((), ())),
                           preferred_element_type=jnp.float32)


def _dense_body(f1_ref, fP_ref, rP_ref, tk4_ref,
                q1w_ref, q1b_ref, q2w_ref, q2b_ref,
                g1w4_ref, g1b4_ref, g2w4_ref, g2b4_ref,
                k14_ref, k24_ref, v14_ref, v24_ref,
                d1_ref, d2_ref,
                selw_ref, sel4_ref, tsel_ref, g32_ref,
                out_ref):
    h = _elu(_dot(f1_ref[...], q1w_ref[...]) + q1b_ref[...])
    Q = _dot(h, q2w_ref[...]) + q2b_ref[...]

    rP = rP_ref[...]
    fP = fP_ref[...]

    wgtP = _dot(rP, selw_ref[...])
    geo1 = _elu(_dot(rP, g1w4_ref[...]) + g1b4_ref[...])
    geoP = (_dot(geo1, g2w4_ref[...]) + g2b4_ref[...]) * fP * wgtP

    K2P = _dot(_elu(_dot(geoP, k14_ref[...])), k24_ref[...])
    VP = _dot(_elu(_dot(geoP, v14_ref[...])), v24_ref[...])

    mz4 = jnp.where(tk4_ref[...] == 0, 0.0, 1.0)
    K2m = K2P * _dot(mz4, sel4_ref[...])

    Qt = _dot(Q, tsel_ref[...])
    K2m3 = K2m.reshape(B, 4, 128)
    g32 = g32_ref[...]
    t4 = None
    for hh in range(NH):
        Qth = Qt[:, 128 * hh:128 * (hh + 1)]
        prod = (K2m3 * Qth[:, None, :]).reshape(B4, 128)
        MqB = _dot(prod, g32)
        wv = MqB * VP
        th = _dot(wv, d1_ref[..., 32 * hh:32 * (hh + 1)])
        t4 = th if t4 is None else t4 + th
    t = jnp.sum(t4.reshape(B, 4, ND), axis=1)
    out_ref[...] = _elu(_dot(_elu(t), d2_ref[...]))


def _row_spec(*shape):
    return pl.BlockSpec(shape, lambda b: (b,) + (0,) * (len(shape) - 1))


def _w_spec(*shape):
    return pl.BlockSpec(shape, lambda b: (0,) * len(shape))


def _tc_dense(f1, fP, rP, tk4, q1w, q1b, q2w, q2b, g1w4, g1b4, g2w4, g2b4,
              k14, k24, v14, v24, d1w, d2w, selw, sel4, tsel, g32):
    grid = (N1 // B,)
    in_specs = [
        _row_spec(B, NI),
        _row_spec(B4, 128),
        _row_spec(B4, 64),
        _row_spec(B4, 4),
        _w_spec(NI, ND), _w_spec(1, ND), _w_spec(ND, NH * ND), _w_spec(1, NH * ND),
        _w_spec(64, 128), _w_spec(1, 128), _w_spec(128, 128), _w_spec(1, 128),
        _w_spec(128, 128), _w_spec(128, 128), _w_spec(128, 128), _w_spec(128, 128),
        _w_spec(128, 128), _w_spec(ND, ND),
        _w_spec(64, 128), _w_spec(4, 128), _w_spec(128, 512), _w_spec(128, 128),
    ]
    return pl.pallas_call(
        _dense_body,
        grid=grid,
        in_specs=in_specs,
        out_specs=_row_spec(B, ND),
        out_shape=jax.ShapeDtypeStruct((N1, ND), jnp.float32),
    )(f1, fP, rP, tk4, q1w, q1b, q2w, q2b, g1w4, g1b4, g2w4, g2b4,
      k14, k24, v14, v24, d1w, d2w, selw, sel4, tsel, g32)


def _blkdiag4(w):
    a, b = w.shape
    z = jnp.zeros((a, b), w.dtype)
    rows = []
    for i in range(4):
        rows.append(jnp.concatenate(
            [w if j == i else z for j in range(4)], axis=1))
    return jnp.concatenate(rows, axis=0)


def _np_const(arr):
    return jnp.asarray(arr, jnp.float32)


def kernel(features1, features2, x1, x2, nuv1, nuv2, topk,
           q1_w, q1_b, q2_w, q2_b, g1_w, g1_b, g2_w, g2_b,
           k1_w, k2_w, v1_w, v2_w, d1_w, d2_w):
    f32 = jnp.float32
    gtab = jnp.concatenate(
        [x2, nuv2.reshape(N2, 9), jnp.zeros((N2, 4), f32)], axis=1)
    ntab = jnp.concatenate(
        [jnp.zeros((N1, 1), f32), x1, nuv1.reshape(N1, 9),
         jnp.zeros((N1, 3), f32)], axis=1)
    ntab = jnp.pad(ntab, ((0, NNODE - N1), (0, 0)))
    idx3 = jnp.pad(topk.reshape(-1), (0, E_PAD - E)).reshape(NW, NCH, CH)
    fE, rE = _sc_gather_rl(features2, gtab, ntab, idx3)
    fP = fE.reshape(E_PAD // 4, 128)
    rP = rE.reshape(E_PAD // 4, 64)

    tk4 = jnp.pad(topk.reshape(E // 4, 4), ((0, E_PAD // 4 - E // 4), (0, 0)))

    g1p = jnp.concatenate([g1_w, jnp.zeros((4, ND), f32)], axis=0)
    g1w4 = _blkdiag4(g1p)
    g1b4 = jnp.tile(g1_b, 4).reshape(1, 128)
    g2w4 = _blkdiag4(g2_w)
    g2b4 = jnp.tile(g2_b, 4).reshape(1, 128)
    k14 = _blkdiag4(k1_w)
    k24 = _blkdiag4(k2_w)
    v14 = _blkdiag4(v1_w)
    v24 = _blkdiag4(v2_w)

    d1p = jnp.concatenate(
        [jnp.tile(d1_w[32 * hh:32 * (hh + 1), :], (4, 1)) for hh in range(4)],
        axis=1)

    selw = np.zeros((64, 128), np.float32)
    for jl in range(4):
        selw[jl * 16 + 12, jl * 32:(jl + 1) * 32] = 1.0
    sel4 = np.zeros((4, 128), np.float32)
    for jl in range(4):
        sel4[jl, jl * 32:(jl + 1) * 32] = 1.0
    tsel = np.zeros((128, 512), np.float32)
    for hh in range(4):
        for jl in range(4):
            for d in range(ND):
                tsel[hh * 32 + d, hh * 128 + jl * 32 + d] = 1.0
    g32 = np.zeros((128, 128), np.float32)
    for jl in range(4):
        g32[jl * 32:(jl + 1) * 32, jl * 32:(jl + 1) * 32] = 1.0

    return _tc_dense(
        features1, fP, rP, tk4,
        q1_w, q1_b.reshape(1, ND), q2_w, q2_b.reshape(1, NH * ND),
        g1w4, g1b4, g2w4, g2b4, k14, k24, v14, v24, d1p, d2_w,
        _np_const(selw), _np_const(sel4), _np_const(tsel),
        _np_const(g32))
